# Initial kernel scaffold; baseline (speedup 1.0000x reference)
import jax
import jax.numpy as jnp
from jax.experimental import pallas as pl


def kernel(t, x, edge_index, edge_weight, distance_to_root, node_physical_distance, kplus, kprimeplus, initial_min, initial_max, W1, b1, W2, b2, W3, b3):
    raise NotImplementedError("write your pallas kernel here")


# speedup vs baseline: 4.9675x; 4.9675x over previous
import functools

import jax
import jax.numpy as jnp
from jax import lax
from jax.experimental import pallas as pl
from jax.experimental.pallas import tpu as pltpu
from jax.experimental.pallas import tpu_sc as plsc

N = 10000
H = 128
E = 320000

NC = 2
NS = 16
NW = NC * NS
L = 16
C = 128
HL = H // L

NPAD = 10240
ROWS_PT = NPAD // NS
EPAD = 327680
EPT = EPAD // NW
NCHUNK = EPT // C

_mesh = plsc.VectorSubcoreMesh(
    core_axis_name="c", subcore_axis_name="s", num_cores=NC, num_subcores=NS
)


@functools.partial(
    pl.kernel,
    out_type=jax.ShapeDtypeStruct((NC, NPAD, L), jnp.float32),
    mesh=_mesh,
    scratch_types=[
        pltpu.VMEM((C,), jnp.int32),
        pltpu.VMEM((C, L), jnp.float32),
        pltpu.VMEM_SHARED((NPAD, L), jnp.float32),
    ],
    compiler_params=pltpu.CompilerParams(use_tc_tiling_on_sc=False),
)
def _sc_deg(dst_hbm, wd_hbm, out_hbm, dst_v, rows_v, acc_sh):
    cid = lax.axis_index("c")
    sid = lax.axis_index("s")
    wid = sid * NC + cid

    def zero_row(i, carry):
        rows_v[i, :] = jnp.zeros((L,), jnp.float32)
        return carry

    lax.fori_loop(0, C, zero_row, 0)
    nbase = sid * ROWS_PT
    for k in range(ROWS_PT // C):
        pltpu.sync_copy(rows_v, acc_sh.at[pl.ds(nbase + k * C, C)])
    plsc.subcore_barrier()

    ebase = wid * EPT

    def chunk(ci, carry):
        off = ebase + ci * C
        pltpu.sync_copy(dst_hbm.at[pl.ds(off, C)], dst_v)
        pltpu.sync_copy(wd_hbm.at[pl.ds(off, C)], rows_v)
        pltpu.sync_copy(rows_v, acc_sh.at[dst_v], add=True)
        return carry

    lax.fori_loop(0, NCHUNK, chunk, 0)
    plsc.subcore_barrier()

    for k in range(ROWS_PT // C):
        pltpu.sync_copy(acc_sh.at[pl.ds(nbase + k * C, C)], rows_v)
        pltpu.sync_copy(rows_v, out_hbm.at[cid, pl.ds(nbase + k * C, C)])


@functools.partial(
    pl.kernel,
    out_type=jax.ShapeDtypeStruct((NC, NPAD, H), jnp.float32),
    mesh=_mesh,
    scratch_types=[
        pltpu.VMEM((C,), jnp.int32),
        pltpu.VMEM((C,), jnp.int32),
        pltpu.VMEM((C, L), jnp.float32),
        pltpu.VMEM((C, H), jnp.float32),
        pltpu.VMEM_SHARED((NPAD, H), jnp.float32),
    ],
    compiler_params=pltpu.CompilerParams(use_tc_tiling_on_sc=False),
)
def _sc_agg(y_hbm, src_hbm, dst_hbm, w_hbm, out_hbm, src_v, dst_v, w_v, rows_v, acc_sh):
    cid = lax.axis_index("c")
    sid = lax.axis_index("s")
    wid = sid * NC + cid

    def zero_row(i, carry):
        for j in range(HL):
            rows_v[i, pl.ds(j * L, L)] = jnp.zeros((L,), jnp.float32)
        return carry

    lax.fori_loop(0, C, zero_row, 0)
    nbase = sid * ROWS_PT
    for k in range(ROWS_PT // C):
        pltpu.sync_copy(rows_v, acc_sh.at[pl.ds(nbase + k * C, C)])
    plsc.subcore_barrier()

    ebase = wid * EPT

    def chunk(ci, carry):
        off = ebase + ci * C
        pltpu.sync_copy(src_hbm.at[pl.ds(off, C)], src_v)
        pltpu.sync_copy(dst_hbm.at[pl.ds(off, C)], dst_v)
        pltpu.sync_copy(w_hbm.at[pl.ds(off, C)], w_v)
        pltpu.sync_copy(y_hbm.at[src_v], rows_v)

        def scale(e, carry2):
            wb = w_v[e, :]
            for j in range(HL):
                rows_v[e, pl.ds(j * L, L)] = rows_v[e, pl.ds(j * L, L)] * wb
            return carry2

        lax.fori_loop(0, C, scale, 0)
        pltpu.sync_copy(rows_v, acc_sh.at[dst_v], add=True)
        return carry

    lax.fori_loop(0, NCHUNK, chunk, 0)
    plsc.subcore_barrier()

    for k in range(ROWS_PT // C):
        pltpu.sync_copy(acc_sh.at[pl.ds(nbase + k * C, C)], rows_v)
        pltpu.sync_copy(rows_v, out_hbm.at[cid, pl.ds(nbase + k * C, C)])


def _dis(d_ref):
    return lax.rsqrt(d_ref[0] + d_ref[1] + 1.0)


def _softplus(x):
    return jnp.maximum(x, 0.0) + jnp.log1p(jnp.exp(-jnp.abs(x)))


def _dot(a, b):
    return jnp.dot(a, b, precision=lax.Precision.HIGHEST,
                   preferred_element_type=jnp.float32)


def _pre_body(h_ref, w_ref, d_ref, y_ref):
    y_ref[...] = _dis(d_ref) * _dot(h_ref[...], w_ref[...])


def _mid_body(p_ref, y_ref, din_ref, dout_ref, b_ref, w_ref, out_ref):
    x = _dis(din_ref) * (p_ref[0] + p_ref[1] + y_ref[...]) + b_ref[...]
    out_ref[...] = _dis(dout_ref) * _dot(_softplus(x), w_ref[...])


def _post_body(p_ref, y_ref, d_ref, b_ref, out_ref):
    out_ref[...] = _dis(d_ref) * (p_ref[0] + p_ref[1] + y_ref[...]) + b_ref[...]


_f32 = jnp.float32
_B = 2000
_ROW = pl.BlockSpec((_B, H), lambda i: (i, 0))
_PART = pl.BlockSpec((NC, _B, H), lambda i: (0, i, 0))
_DEG = pl.BlockSpec((NC, _B, 1), lambda i: (0, i, 0))
_BIAS = pl.BlockSpec((1, H), lambda i: (0, 0))
_WMAT = pl.BlockSpec((H, H), lambda i: (0, 0))

_tc_pre = pl.pallas_call(
    _pre_body, grid=(N // _B,),
    in_specs=[_ROW, _WMAT, _DEG], out_specs=_ROW,
    out_shape=jax.ShapeDtypeStruct((N, H), _f32))
_tc_mid = pl.pallas_call(
    _mid_body, grid=(N // _B,),
    in_specs=[_PART, _ROW, _DEG, _DEG, _BIAS, _WMAT], out_specs=_ROW,
    out_shape=jax.ShapeDtypeStruct((N, H), _f32))
_tc_post = pl.pallas_call(
    _post_body, grid=(N // _B,),
    in_specs=[_PART, _ROW, _DEG, _BIAS], out_specs=_ROW,
    out_shape=jax.ShapeDtypeStruct((N, H), _f32))


def kernel(t, x, edge_index, edge_weight, distance_to_root, node_physical_distance,
           kplus, kprimeplus, initial_min, initial_max, W1, b1, W2, b2, W3, b3):
    h0 = jnp.concatenate(
        [x, distance_to_root, node_physical_distance, kplus, kprimeplus,
         initial_min, initial_max], axis=1)

    pad = EPAD - E
    src_p = jnp.concatenate([edge_index[0], jnp.zeros((pad,), jnp.int32)])
    dst_p = jnp.concatenate([edge_index[1], jnp.zeros((pad,), jnp.int32)])
    w12_1 = jnp.concatenate([edge_weight, jnp.zeros((pad,), _f32)])
    w3_1 = jnp.concatenate([jnp.ones((E,), _f32), jnp.zeros((pad,), _f32)])
    w12 = jnp.broadcast_to(w12_1[:, None], (EPAD, L))
    w3 = jnp.broadcast_to(w3_1[:, None], (EPAD, L))
    wd = jnp.concatenate([w12[:, :8], w3[:, :8]], axis=1)

    degp = _sc_deg(dst_p, wd)
    d12 = degp[:, :N, 0:1]
    d3 = degp[:, :N, 8:9]

    b1r, b2r, b3r = (b.reshape(1, H) for b in (b1, b2, b3))

    y1 = _tc_pre(h0, W1, d12)
    p1 = _sc_agg(y1, src_p, dst_p, w12)
    y2 = _tc_mid(p1[:, :N], y1, d12, d12, b1r, W2)
    p2 = _sc_agg(y2, src_p, dst_p, w12)
    y3 = _tc_mid(p2[:, :N], y2, d12, d3, b2r, W3)
    p3 = _sc_agg(y3, src_p, dst_p, w3)
    return _tc_post(p3[:, :N], y3, d3, b3r)


# speedup vs baseline: 6.3020x; 1.2686x over previous
import functools

import jax
import jax.numpy as jnp
from jax import lax
from jax.experimental import pallas as pl
from jax.experimental.pallas import tpu as pltpu
from jax.experimental.pallas import tpu_sc as plsc

N = 10000
H = 128
E = 320000

NC = 2
NS = 16
NW = NC * NS
L = 16
C = 128
HL = H // L

NPAD = 10240
ROWS_PT = NPAD // NS
EPAD = 327680
EPT = EPAD // NW
NCHUNK = EPT // C

_mesh = plsc.VectorSubcoreMesh(
    core_axis_name="c", subcore_axis_name="s", num_cores=NC, num_subcores=NS
)


@functools.partial(
    pl.kernel,
    out_type=jax.ShapeDtypeStruct((NC, NPAD, L), jnp.float32),
    mesh=_mesh,
    scratch_types=[
        pltpu.VMEM((C,), jnp.int32),
        pltpu.VMEM((C, L), jnp.float32),
        pltpu.VMEM_SHARED((NPAD, L), jnp.float32),
    ],
    compiler_params=pltpu.CompilerParams(use_tc_tiling_on_sc=False),
)
def _sc_deg(dst_hbm, wd_hbm, out_hbm, dst_v, rows_v, acc_sh):
    cid = lax.axis_index("c")
    sid = lax.axis_index("s")
    wid = sid * NC + cid

    def zero_row(i, carry):
        rows_v[i, :] = jnp.zeros((L,), jnp.float32)
        return carry

    lax.fori_loop(0, C, zero_row, 0)
    nbase = sid * ROWS_PT
    for k in range(ROWS_PT // C):
        pltpu.sync_copy(rows_v, acc_sh.at[pl.ds(nbase + k * C, C)])
    plsc.subcore_barrier()

    ebase = wid * EPT

    def chunk(ci, carry):
        off = ebase + ci * C
        pltpu.sync_copy(dst_hbm.at[pl.ds(off, C)], dst_v)
        pltpu.sync_copy(wd_hbm.at[pl.ds(off, C)], rows_v)
        pltpu.sync_copy(rows_v, acc_sh.at[dst_v], add=True)
        return carry

    lax.fori_loop(0, NCHUNK, chunk, 0)
    plsc.subcore_barrier()

    for k in range(ROWS_PT // C):
        pltpu.sync_copy(acc_sh.at[pl.ds(nbase + k * C, C)], rows_v)
        pltpu.sync_copy(rows_v, out_hbm.at[cid, pl.ds(nbase + k * C, C)])


SK = 4
NSUP = NCHUNK // SK
EC = EPAD // C
CHUNK_BYTES = C * H * 4


def _agg_body(y_hbm, sd_hbm, w_hbm, out_hbm, sd_v, w_v, rows_v, acc_sh,
              gsem, ssem):
    cid = lax.axis_index("c")
    sid = lax.axis_index("s")
    wid = sid * NC + cid

    def zero_row(i, carry):
        for j in range(HL):
            rows_v[0, i, pl.ds(j * L, L)] = jnp.zeros((L,), jnp.float32)
            rows_v[1, i, pl.ds(j * L, L)] = jnp.zeros((L,), jnp.float32)
        return carry

    lax.fori_loop(0, C, zero_row, 0)
    nbase = sid * ROWS_PT
    for k in range(ROWS_PT // C):
        pltpu.sync_copy(rows_v.at[0], acc_sh.at[pl.ds(nbase + k * C, C)])
    plsc.subcore_barrier()

    rbase = wid * (EPT // C)

    def load_idx(s, b):
        row = rbase + s * SK
        pltpu.sync_copy(sd_hbm.at[pl.ds(row, SK)], sd_v.at[b])
        pltpu.sync_copy(w_hbm.at[pl.ds(row, SK)], w_v)

    def fire_gather(s, b, j, p):
        pltpu.async_copy(y_hbm.at[sd_v.at[b, j, 0]], rows_v.at[p], gsem)

    def wait_gather(b, j, p):
        pltpu.make_async_copy(y_hbm.at[sd_v.at[b, j, 0]], rows_v.at[p],
                              gsem).wait()

    def fire_scatter(b, j, p):
        pltpu.async_copy(rows_v.at[p], acc_sh.at[sd_v.at[b, j, 1]], ssem,
                         add=True)

    def wait_scatter(p):
        pltpu.make_async_copy(rows_v.at[p], acc_sh.at[sd_v.at[0, 0, 1]],
                              ssem).wait()

    def scale(b, j, p):
        @plsc.parallel_loop(0, C, unroll=4)
        def _scale(e):
            wb = w_v[j, e, :]
            for k in range(HL):
                rows_v[p, e, pl.ds(k * L, L)] = (
                    rows_v[p, e, pl.ds(k * L, L)] * wb)

    load_idx(0, 0)
    fire_gather(0, 0, 0, 0)
    fire_scatter(0, 0, 1)

    def super_chunk(s, carry):
        b = s & 1
        nb = b ^ 1
        last = s == NSUP - 1
        ns = jnp.where(last, s, s + 1)
        for j in range(SK):
            p = j & 1
            wait_gather(b, j, p)
            wait_scatter(p ^ 1)
            if j < SK - 1:
                fire_gather(s, b, j + 1, p ^ 1)
                scale(b, j, p)
                fire_scatter(b, j, p)
            else:
                scale(b, j, p)
                fire_scatter(b, j, p)
                load_idx(ns, nb)
                fire_gather(ns, nb, 0, p ^ 1)
        return carry

    lax.fori_loop(0, NSUP, super_chunk, 0)
    wait_scatter(0)
    pltpu.make_async_copy(y_hbm.at[sd_v.at[0, 0, 0]], rows_v.at[0],
                          gsem).wait()
    plsc.subcore_barrier()

    for k in range(ROWS_PT // C):
        pltpu.sync_copy(acc_sh.at[pl.ds(nbase + k * C, C)], rows_v.at[0])
        pltpu.sync_copy(rows_v.at[0], out_hbm.at[cid, pl.ds(nbase + k * C, C)])


_sc_agg_w = pl.kernel(
    _agg_body,
    out_type=jax.ShapeDtypeStruct((NC, NPAD, H), jnp.float32),
    mesh=_mesh,
    scratch_types=[
        pltpu.VMEM((2, SK, 2, C), jnp.int32),
        pltpu.VMEM((SK, C, L), jnp.float32),
        pltpu.VMEM((2, C, H), jnp.float32),
        pltpu.VMEM_SHARED((NPAD, H), jnp.float32),
        pltpu.SemaphoreType.DMA,
        pltpu.SemaphoreType.DMA,
    ],
    compiler_params=pltpu.CompilerParams(use_tc_tiling_on_sc=False),
)


def _dis(d_ref):
    return lax.rsqrt(d_ref[0] + d_ref[1] + 1.0)


def _softplus(x):
    return jnp.maximum(x, 0.0) + jnp.log1p(jnp.exp(-jnp.abs(x)))


def _dot(a, b):
    return jnp.dot(a, b, precision=lax.Precision.HIGHEST,
                   preferred_element_type=jnp.float32)


def _pre_body(h_ref, w_ref, d_ref, y_ref):
    y_ref[...] = _dis(d_ref) * _dot(h_ref[...], w_ref[...])


def _mid_body(p_ref, y_ref, din_ref, dout_ref, b_ref, w_ref, out_ref):
    x = _dis(din_ref) * (p_ref[0] + p_ref[1] + y_ref[...]) + b_ref[...]
    out_ref[...] = _dis(dout_ref) * _dot(_softplus(x), w_ref[...])


def _post_body(p_ref, y_ref, d_ref, b_ref, out_ref):
    out_ref[...] = _dis(d_ref) * (p_ref[0] + p_ref[1] + y_ref[...]) + b_ref[...]


_f32 = jnp.float32
_B = 2000
_ROW = pl.BlockSpec((_B, H), lambda i: (i, 0))
_PART = pl.BlockSpec((NC, _B, H), lambda i: (0, i, 0))
_DEG = pl.BlockSpec((NC, _B, 1), lambda i: (0, i, 0))
_BIAS = pl.BlockSpec((1, H), lambda i: (0, 0))
_WMAT = pl.BlockSpec((H, H), lambda i: (0, 0))

_tc_pre = pl.pallas_call(
    _pre_body, grid=(N // _B,),
    in_specs=[_ROW, _WMAT, _DEG], out_specs=_ROW,
    out_shape=jax.ShapeDtypeStruct((N, H), _f32))
_tc_mid = pl.pallas_call(
    _mid_body, grid=(N // _B,),
    in_specs=[_PART, _ROW, _DEG, _DEG, _BIAS, _WMAT], out_specs=_ROW,
    out_shape=jax.ShapeDtypeStruct((N, H), _f32))
_tc_post = pl.pallas_call(
    _post_body, grid=(N // _B,),
    in_specs=[_PART, _ROW, _DEG, _BIAS], out_specs=_ROW,
    out_shape=jax.ShapeDtypeStruct((N, H), _f32))


def kernel(t, x, edge_index, edge_weight, distance_to_root, node_physical_distance,
           kplus, kprimeplus, initial_min, initial_max, W1, b1, W2, b2, W3, b3):
    h0 = jnp.concatenate(
        [x, distance_to_root, node_physical_distance, kplus, kprimeplus,
         initial_min, initial_max], axis=1)

    pad = EPAD - E
    src_p = jnp.concatenate([edge_index[0], jnp.zeros((pad,), jnp.int32)])
    dst_p = jnp.concatenate([edge_index[1], jnp.full((pad,), N, jnp.int32)])
    w12_1 = jnp.concatenate([edge_weight, jnp.zeros((pad,), _f32)])
    w12 = jnp.broadcast_to(w12_1[:, None], (EPAD, L))
    wd = jnp.concatenate([w12[:, :8], jnp.ones((EPAD, 8), _f32)], axis=1)

    degp = _sc_deg(dst_p, wd)
    sd2d = jnp.stack([src_p.reshape(EC, C), dst_p.reshape(EC, C)], axis=1)
    w12_3d = w12.reshape(EC, C, L)
    d12 = degp[:, :N, 0:1]
    d3 = degp[:, :N, 8:9]

    b1r, b2r, b3r = (b.reshape(1, H) for b in (b1, b2, b3))

    y1 = _tc_pre(h0, W1, d12)
    p1 = _sc_agg_w(y1, sd2d, w12_3d)
    y2 = _tc_mid(p1[:, :N], y1, d12, d12, b1r, W2)
    p2 = _sc_agg_w(y2, sd2d, w12_3d)
    y3 = _tc_mid(p2[:, :N], y2, d12, d3, b2r, W3)
    p3 = _sc_agg_w(y3, sd2d, jnp.ones((EC, C, L), _f32))
    return _tc_post(p3[:, :N], y3, d3, b3r)


# speedup vs baseline: 9.9926x; 1.5856x over previous
import functools

import jax
import jax.numpy as jnp
from jax import lax
from jax.experimental import pallas as pl
from jax.experimental.pallas import tpu as pltpu
from jax.experimental.pallas import tpu_sc as plsc

N = 10000
H = 128
E = 320000

NC = 2
NS = 16
NW = NC * NS
L = 16
C = 128
HL = H // L

NPAD = 10240
ROWS_PT = NPAD // NS
EPAD = 327680
EPT = EPAD // NW
NCHUNK = EPT // C

_mesh = plsc.VectorSubcoreMesh(
    core_axis_name="c", subcore_axis_name="s", num_cores=NC, num_subcores=NS
)


@functools.partial(
    pl.kernel,
    out_type=jax.ShapeDtypeStruct((NC, NPAD, L), jnp.float32),
    mesh=_mesh,
    scratch_types=[
        pltpu.VMEM((C,), jnp.int32),
        pltpu.VMEM((C, L), jnp.float32),
        pltpu.VMEM_SHARED((NPAD, L), jnp.float32),
    ],
    compiler_params=pltpu.CompilerParams(use_tc_tiling_on_sc=False),
)
def _sc_deg(dst_hbm, wd_hbm, out_hbm, dst_v, rows_v, acc_sh):
    cid = lax.axis_index("c")
    sid = lax.axis_index("s")
    wid = sid * NC + cid

    def zero_row(i, carry):
        rows_v[i, :] = jnp.zeros((L,), jnp.float32)
        return carry

    lax.fori_loop(0, C, zero_row, 0)
    nbase = sid * ROWS_PT
    for k in range(ROWS_PT // C):
        pltpu.sync_copy(rows_v, acc_sh.at[pl.ds(nbase + k * C, C)])
    plsc.subcore_barrier()

    ebase = wid * EPT

    def chunk(ci, carry):
        off = ebase + ci * C
        pltpu.sync_copy(dst_hbm.at[pl.ds(off, C)], dst_v)
        pltpu.sync_copy(wd_hbm.at[pl.ds(off, C)], rows_v)
        pltpu.sync_copy(rows_v, acc_sh.at[dst_v], add=True)
        return carry

    lax.fori_loop(0, NCHUNK, chunk, 0)
    plsc.subcore_barrier()

    for k in range(ROWS_PT // C):
        pltpu.sync_copy(acc_sh.at[pl.ds(nbase + k * C, C)], rows_v)
        pltpu.sync_copy(rows_v, out_hbm.at[cid, pl.ds(nbase + k * C, C)])


SK = 4
HH = H // 2
HL2 = HH // L
EPT2 = EPAD // NS
NCHUNK2 = EPT2 // C
NSUP = NCHUNK2 // SK
EC = EPAD // C


def _agg_body(yh_hbm, sd_hbm, w_hbm, out_hbm, sd_v, w_v, rows_v, y_sh,
              acc_sh, gsem, ssem):
    cid = lax.axis_index("c")
    sid = lax.axis_index("s")

    def zero_row(i, carry):
        for j in range(HL2):
            rows_v[0, i, pl.ds(j * L, L)] = jnp.zeros((L,), jnp.float32)
            rows_v[1, i, pl.ds(j * L, L)] = jnp.zeros((L,), jnp.float32)
        return carry

    lax.fori_loop(0, C, zero_row, 0)
    nbase = sid * ROWS_PT
    for k in range(ROWS_PT // C):
        pltpu.sync_copy(rows_v.at[0], acc_sh.at[pl.ds(nbase + k * C, C)])
        pltpu.sync_copy(yh_hbm.at[cid, pl.ds(nbase + k * C, C)],
                        y_sh.at[pl.ds(nbase + k * C, C)])
    plsc.subcore_barrier()

    rbase = sid * NCHUNK2

    def load_idx(s, b):
        row = rbase + s * SK
        pltpu.sync_copy(sd_hbm.at[pl.ds(row, SK)], sd_v.at[b])
        pltpu.sync_copy(w_hbm.at[pl.ds(row, SK)], w_v)

    def fire_gather(s, b, j, p):
        pltpu.async_copy(y_sh.at[sd_v.at[b, j, 0]], rows_v.at[p], gsem)

    def wait_gather(b, j, p):
        pltpu.make_async_copy(y_sh.at[sd_v.at[b, j, 0]], rows_v.at[p],
                              gsem).wait()

    def fire_scatter(b, j, p):
        pltpu.async_copy(rows_v.at[p], acc_sh.at[sd_v.at[b, j, 1]], ssem,
                         add=True)

    def wait_scatter(p):
        pltpu.make_async_copy(rows_v.at[p], acc_sh.at[sd_v.at[0, 0, 1]],
                              ssem).wait()

    def scale(b, j, p):
        @plsc.parallel_loop(0, C, unroll=4)
        def _scale(e):
            wb = w_v[j, e, :]
            for k in range(HL2):
                rows_v[p, e, pl.ds(k * L, L)] = (
                    rows_v[p, e, pl.ds(k * L, L)] * wb)

    load_idx(0, 0)
    fire_gather(0, 0, 0, 0)
    fire_scatter(0, 0, 1)

    def super_chunk(s, carry):
        b = s & 1
        nb = b ^ 1
        last = s == NSUP - 1
        ns = jnp.where(last, s, s + 1)
        for j in range(SK):
            p = j & 1
            wait_gather(b, j, p)
            wait_scatter(p ^ 1)
            if j < SK - 1:
                fire_gather(s, b, j + 1, p ^ 1)
                scale(b, j, p)
                fire_scatter(b, j, p)
            else:
                scale(b, j, p)
                fire_scatter(b, j, p)
                load_idx(ns, nb)
                fire_gather(ns, nb, 0, p ^ 1)
        return carry

    lax.fori_loop(0, NSUP, super_chunk, 0)
    wait_scatter(0)
    pltpu.make_async_copy(y_sh.at[sd_v.at[0, 0, 0]], rows_v.at[0],
                          gsem).wait()
    plsc.subcore_barrier()

    for k in range(ROWS_PT // C):
        pltpu.sync_copy(acc_sh.at[pl.ds(nbase + k * C, C)], rows_v.at[0])
        pltpu.sync_copy(rows_v.at[0], out_hbm.at[cid, pl.ds(nbase + k * C, C)])


_sc_agg_w = pl.kernel(
    _agg_body,
    out_type=jax.ShapeDtypeStruct((NC, NPAD, HH), jnp.float32),
    mesh=_mesh,
    scratch_types=[
        pltpu.VMEM((2, SK, 2, C), jnp.int32),
        pltpu.VMEM((SK, C, L), jnp.float32),
        pltpu.VMEM((2, C, HH), jnp.float32),
        pltpu.VMEM_SHARED((NPAD, HH), jnp.float32),
        pltpu.VMEM_SHARED((NPAD, HH), jnp.float32),
        pltpu.SemaphoreType.DMA,
        pltpu.SemaphoreType.DMA,
    ],
    compiler_params=pltpu.CompilerParams(use_tc_tiling_on_sc=False),
)


def _dis(d_ref):
    return lax.rsqrt(d_ref[0] + d_ref[1] + 1.0)


def _softplus(x):
    return jnp.maximum(x, 0.0) + jnp.log1p(jnp.exp(-jnp.abs(x)))


def _dot(a, b):
    return jnp.dot(a, b, precision=lax.Precision.HIGHEST,
                   preferred_element_type=jnp.float32)


def _halves(r):
    return jnp.concatenate([r[0], r[1]], axis=-1)


def _pre_body(h_ref, w_ref, d_ref, yh_ref):
    y = _dis(d_ref) * _dot(h_ref[...], w_ref[...])
    yh_ref[0] = y[:, :HH]
    yh_ref[1] = y[:, HH:]


def _mid_body(p_ref, yh_ref, din_ref, dout_ref, b_ref, w_ref, out_ref):
    x = _dis(din_ref) * (_halves(p_ref) + _halves(yh_ref)) + b_ref[...]
    y = _dis(dout_ref) * _dot(_softplus(x), w_ref[...])
    out_ref[0] = y[:, :HH]
    out_ref[1] = y[:, HH:]


def _post_body(p_ref, yh_ref, d_ref, b_ref, out_ref):
    out_ref[...] = _dis(d_ref) * (_halves(p_ref) + _halves(yh_ref)) + b_ref[...]


_f32 = jnp.float32
_B = 640
_ROW = pl.BlockSpec((_B, H), lambda i: (i, 0))
_HALF = pl.BlockSpec((NC, _B, HH), lambda i: (0, i, 0))
_DEG = pl.BlockSpec((NC, _B, 1), lambda i: (0, i, 0))
_BIAS = pl.BlockSpec((1, H), lambda i: (0, 0))
_WMAT = pl.BlockSpec((H, H), lambda i: (0, 0))
_HSH = jax.ShapeDtypeStruct((NC, NPAD, HH), _f32)

_tc_pre = pl.pallas_call(
    _pre_body, grid=(NPAD // _B,),
    in_specs=[_ROW, _WMAT, _DEG], out_specs=_HALF, out_shape=_HSH)
_tc_mid = pl.pallas_call(
    _mid_body, grid=(NPAD // _B,),
    in_specs=[_HALF, _HALF, _DEG, _DEG, _BIAS, _WMAT], out_specs=_HALF,
    out_shape=_HSH)
_tc_post = pl.pallas_call(
    _post_body, grid=(NPAD // _B,),
    in_specs=[_HALF, _HALF, _DEG, _BIAS], out_specs=_ROW,
    out_shape=jax.ShapeDtypeStruct((NPAD, H), _f32))


def kernel(t, x, edge_index, edge_weight, distance_to_root, node_physical_distance,
           kplus, kprimeplus, initial_min, initial_max, W1, b1, W2, b2, W3, b3):
    h0 = jnp.concatenate(
        [x, distance_to_root, node_physical_distance, kplus, kprimeplus,
         initial_min, initial_max], axis=1)
    h0p = jnp.concatenate([h0, jnp.zeros((NPAD - N, H), _f32)])

    pad = EPAD - E
    src_p = jnp.concatenate([edge_index[0], jnp.zeros((pad,), jnp.int32)])
    dst_p = jnp.concatenate([edge_index[1], jnp.full((pad,), N, jnp.int32)])
    w12_1 = jnp.concatenate([edge_weight, jnp.zeros((pad,), _f32)])
    w12 = jnp.broadcast_to(w12_1[:, None], (EPAD, L))
    wd = jnp.concatenate([w12[:, :8], jnp.ones((EPAD, 8), _f32)], axis=1)

    degp = _sc_deg(dst_p, wd)
    sd2d = jnp.stack([src_p.reshape(EC, C), dst_p.reshape(EC, C)], axis=1)
    w12_3d = w12.reshape(EC, C, L)
    d12 = degp[:, :, 0:1]
    d3 = degp[:, :, 8:9]

    b1r, b2r, b3r = (b.reshape(1, H) for b in (b1, b2, b3))

    y1 = _tc_pre(h0p, W1, d12)
    p1 = _sc_agg_w(y1, sd2d, w12_3d)
    y2 = _tc_mid(p1, y1, d12, d12, b1r, W2)
    p2 = _sc_agg_w(y2, sd2d, w12_3d)
    y3 = _tc_mid(p2, y2, d12, d3, b2r, W3)
    p3 = _sc_agg_w(y3, sd2d, jnp.ones((EC, C, L), _f32))
    return _tc_post(p3, y3, d3, b3r)[:N]


# speedup vs baseline: 15.8773x; 1.5889x over previous
import functools

import jax
import jax.numpy as jnp
from jax import lax
from jax.experimental import pallas as pl
from jax.experimental.pallas import tpu as pltpu
from jax.experimental.pallas import tpu_sc as plsc

N = 10000
H = 128
E = 320000

NC = 2
NS = 16
NW = NC * NS
L = 16
C = 128
HL = H // L

NPAD = 10240
ROWS_PT = NPAD // NS
EPAD = 327680
EPT = EPAD // NW
NCHUNK = EPT // C

_mesh = plsc.VectorSubcoreMesh(
    core_axis_name="c", subcore_axis_name="s", num_cores=NC, num_subcores=NS
)


@functools.partial(
    pl.kernel,
    out_type=jax.ShapeDtypeStruct((NC, NPAD, L), jnp.float32),
    mesh=_mesh,
    scratch_types=[
        pltpu.VMEM((C,), jnp.int32),
        pltpu.VMEM((C,), jnp.float32),
        pltpu.VMEM((C, L), jnp.float32),
        pltpu.VMEM_SHARED((NPAD, L), jnp.float32),
    ],
    compiler_params=pltpu.CompilerParams(use_tc_tiling_on_sc=False),
)
def _sc_deg(dst_hbm, w_hbm, out_hbm, dst_v, w_sc, rows_v, acc_sh):
    cid = lax.axis_index("c")
    sid = lax.axis_index("s")
    wid = sid * NC + cid
    lo8 = lax.broadcasted_iota(jnp.int32, (L,), 0) < 8

    def zero_row(i, carry):
        rows_v[i, :] = jnp.zeros((L,), jnp.float32)
        return carry

    lax.fori_loop(0, C, zero_row, 0)
    nbase = sid * ROWS_PT
    for k in range(ROWS_PT // C):
        pltpu.sync_copy(rows_v, acc_sh.at[pl.ds(nbase + k * C, C)])
    plsc.subcore_barrier()

    ebase = wid * EPT

    def chunk(ci, carry):
        off = ebase + ci * C
        pltpu.sync_copy(dst_hbm.at[pl.ds(off, C)], dst_v)
        pltpu.sync_copy(w_hbm.at[pl.ds(off, C)], w_sc)

        @plsc.parallel_loop(0, C // L, unroll=2)
        def _build(g):
            wvec = w_sc[pl.ds(g * L, L)]
            for i in range(L):
                rows_v[g * L + i, :] = jnp.where(lo8, wvec[i],
                                                 jnp.float32(1.0))

        pltpu.sync_copy(rows_v, acc_sh.at[dst_v], add=True)
        return carry

    lax.fori_loop(0, NCHUNK, chunk, 0)
    plsc.subcore_barrier()

    for k in range(ROWS_PT // C):
        pltpu.sync_copy(acc_sh.at[pl.ds(nbase + k * C, C)], rows_v)
        pltpu.sync_copy(rows_v, out_hbm.at[cid, pl.ds(nbase + k * C, C)])


SK = 4
HH = H // 2
HL2 = HH // L
EPT2 = EPAD // NS
NCHUNK2 = EPT2 // C
NSUP = NCHUNK2 // SK
EC = EPAD // C


def _make_agg_body(weighted):
    def body(*args):
        if weighted:
            (yh_hbm, sd_hbm, w_hbm, out_hbm, sd_v, w_v, rows_v, y_sh,
             acc_sh, gsem, ssem) = args
        else:
            (yh_hbm, sd_hbm, out_hbm, sd_v, rows_v, y_sh,
             acc_sh, gsem, ssem) = args
        cid = lax.axis_index("c")
        sid = lax.axis_index("s")

        def zero_row(i, carry):
            for j in range(HL2):
                rows_v[0, i, pl.ds(j * L, L)] = jnp.zeros((L,), jnp.float32)
                rows_v[1, i, pl.ds(j * L, L)] = jnp.zeros((L,), jnp.float32)
            return carry

        lax.fori_loop(0, C, zero_row, 0)
        nbase = sid * ROWS_PT
        for k in range(ROWS_PT // C):
            pltpu.sync_copy(rows_v.at[0], acc_sh.at[pl.ds(nbase + k * C, C)])
            pltpu.sync_copy(yh_hbm.at[cid, pl.ds(nbase + k * C, C)],
                            y_sh.at[pl.ds(nbase + k * C, C)])
        plsc.subcore_barrier()

        rbase = sid * NCHUNK2

        def load_idx(s, b):
            row = rbase + s * SK
            pltpu.sync_copy(sd_hbm.at[pl.ds(row, SK)], sd_v.at[b])
            if weighted:
                pltpu.sync_copy(w_hbm.at[pl.ds(row, SK)], w_v)

        def fire_gather(s, b, j, p):
            pltpu.async_copy(y_sh.at[sd_v.at[b, j, 0]], rows_v.at[p], gsem)

        def wait_gather(b, j, p):
            pltpu.make_async_copy(y_sh.at[sd_v.at[b, j, 0]], rows_v.at[p],
                                  gsem).wait()

        def fire_scatter(b, j, p):
            pltpu.async_copy(rows_v.at[p], acc_sh.at[sd_v.at[b, j, 1]], ssem,
                             add=True)

        def wait_scatter(p):
            pltpu.make_async_copy(rows_v.at[p], acc_sh.at[sd_v.at[0, 0, 1]],
                                  ssem).wait()

        def scale(b, j, p):
            if not weighted:
                return

            @plsc.parallel_loop(0, C // L, unroll=2)
            def _scale(g):
                wvec = w_v[j, pl.ds(g * L, L)]
                for i in range(L):
                    wb = wvec[i]
                    for k in range(HL2):
                        e = g * L + i
                        rows_v[p, e, pl.ds(k * L, L)] = (
                            rows_v[p, e, pl.ds(k * L, L)] * wb)

        load_idx(0, 0)
        fire_gather(0, 0, 0, 0)
        fire_scatter(0, 0, 1)

        def super_chunk(s, carry):
            b = s & 1
            nb = b ^ 1
            last = s == NSUP - 1
            ns = jnp.where(last, s, s + 1)
            for j in range(SK):
                p = j & 1
                wait_gather(b, j, p)
                wait_scatter(p ^ 1)
                if j < SK - 1:
                    fire_gather(s, b, j + 1, p ^ 1)
                    scale(b, j, p)
                    fire_scatter(b, j, p)
                else:
                    scale(b, j, p)
                    fire_scatter(b, j, p)
                    load_idx(ns, nb)
                    fire_gather(ns, nb, 0, p ^ 1)
            return carry

        lax.fori_loop(0, NSUP, super_chunk, 0)
        wait_scatter(0)
        pltpu.make_async_copy(y_sh.at[sd_v.at[0, 0, 0]], rows_v.at[0],
                              gsem).wait()
        plsc.subcore_barrier()

        for k in range(ROWS_PT // C):
            pltpu.sync_copy(acc_sh.at[pl.ds(nbase + k * C, C)], rows_v.at[0])
            pltpu.sync_copy(rows_v.at[0],
                            out_hbm.at[cid, pl.ds(nbase + k * C, C)])

    return body


def _agg_scratch(weighted):
    sc = [
        pltpu.VMEM((2, SK, 2, C), jnp.int32),
        pltpu.VMEM((SK, C), jnp.float32),
        pltpu.VMEM((2, C, HH), jnp.float32),
        pltpu.VMEM_SHARED((NPAD, HH), jnp.float32),
        pltpu.VMEM_SHARED((NPAD, HH), jnp.float32),
        pltpu.SemaphoreType.DMA,
        pltpu.SemaphoreType.DMA,
    ]
    if not weighted:
        del sc[1]
    return sc


_sc_agg_w = pl.kernel(
    _make_agg_body(True),
    out_type=jax.ShapeDtypeStruct((NC, NPAD, HH), jnp.float32),
    mesh=_mesh,
    scratch_types=_agg_scratch(True),
    compiler_params=pltpu.CompilerParams(use_tc_tiling_on_sc=False),
)

_sc_agg_u = pl.kernel(
    _make_agg_body(False),
    out_type=jax.ShapeDtypeStruct((NC, NPAD, HH), jnp.float32),
    mesh=_mesh,
    scratch_types=_agg_scratch(False),
    compiler_params=pltpu.CompilerParams(use_tc_tiling_on_sc=False),
)


def _dis(d_ref):
    return lax.rsqrt(d_ref[0] + d_ref[1] + 1.0)


def _softplus(x):
    return jnp.maximum(x, 0.0) + jnp.log1p(jnp.exp(-jnp.abs(x)))


def _dot(a, b):
    return jnp.dot(a, b, precision=lax.Precision.HIGHEST,
                   preferred_element_type=jnp.float32)


def _halves(r):
    return jnp.concatenate([r[0], r[1]], axis=-1)


def _pre_body(h_ref, w_ref, d_ref, yh_ref):
    y = _dis(d_ref) * _dot(h_ref[...], w_ref[...])
    yh_ref[0] = y[:, :HH]
    yh_ref[1] = y[:, HH:]


def _mid_body(p_ref, yh_ref, din_ref, dout_ref, b_ref, w_ref, out_ref):
    x = _dis(din_ref) * (_halves(p_ref) + _halves(yh_ref)) + b_ref[...]
    y = _dis(dout_ref) * _dot(_softplus(x), w_ref[...])
    out_ref[0] = y[:, :HH]
    out_ref[1] = y[:, HH:]


def _post_body(p_ref, yh_ref, d_ref, b_ref, out_ref):
    out_ref[...] = _dis(d_ref) * (_halves(p_ref) + _halves(yh_ref)) + b_ref[...]


_f32 = jnp.float32
_B = 640
_ROW = pl.BlockSpec((_B, H), lambda i: (i, 0))
_HALF = pl.BlockSpec((NC, _B, HH), lambda i: (0, i, 0))
_DEG = pl.BlockSpec((NC, _B, 1), lambda i: (0, i, 0))
_BIAS = pl.BlockSpec((1, H), lambda i: (0, 0))
_WMAT = pl.BlockSpec((H, H), lambda i: (0, 0))
_HSH = jax.ShapeDtypeStruct((NC, NPAD, HH), _f32)

_tc_pre = pl.pallas_call(
    _pre_body, grid=(NPAD // _B,),
    in_specs=[_ROW, _WMAT, _DEG], out_specs=_HALF, out_shape=_HSH)
_tc_mid = pl.pallas_call(
    _mid_body, grid=(NPAD // _B,),
    in_specs=[_HALF, _HALF, _DEG, _DEG, _BIAS, _WMAT], out_specs=_HALF,
    out_shape=_HSH)
_tc_post = pl.pallas_call(
    _post_body, grid=(NPAD // _B,),
    in_specs=[_HALF, _HALF, _DEG, _BIAS], out_specs=_ROW,
    out_shape=jax.ShapeDtypeStruct((NPAD, H), _f32))


def kernel(t, x, edge_index, edge_weight, distance_to_root, node_physical_distance,
           kplus, kprimeplus, initial_min, initial_max, W1, b1, W2, b2, W3, b3):
    h0 = jnp.concatenate(
        [x, distance_to_root, node_physical_distance, kplus, kprimeplus,
         initial_min, initial_max], axis=1)
    h0p = jnp.concatenate([h0, jnp.zeros((NPAD - N, H), _f32)])

    pad = EPAD - E
    src_p = jnp.concatenate([edge_index[0], jnp.zeros((pad,), jnp.int32)])
    dst_p = jnp.concatenate([edge_index[1], jnp.full((pad,), N, jnp.int32)])
    w12_1 = jnp.concatenate([edge_weight, jnp.zeros((pad,), _f32)])

    degp = _sc_deg(dst_p, w12_1)
    sd2d = jnp.stack([src_p.reshape(EC, C), dst_p.reshape(EC, C)], axis=1)
    w2d = w12_1.reshape(EC, C)
    d12 = degp[:, :, 0:1]
    d3 = degp[:, :, 8:9]

    b1r, b2r, b3r = (b.reshape(1, H) for b in (b1, b2, b3))

    y1 = _tc_pre(h0p, W1, d12)
    p1 = _sc_agg_w(y1, sd2d, w2d)
    y2 = _tc_mid(p1, y1, d12, d12, b1r, W2)
    p2 = _sc_agg_w(y2, sd2d, w2d)
    y3 = _tc_mid(p2, y2, d12, d3, b2r, W3)
    p3 = _sc_agg_u(y3, sd2d)
    return _tc_post(p3, y3, d3, b3r)[:N]


# speedup vs baseline: 16.0211x; 1.0091x over previous
import functools

import jax
import jax.numpy as jnp
from jax import lax
from jax.experimental import pallas as pl
from jax.experimental.pallas import tpu as pltpu
from jax.experimental.pallas import tpu_sc as plsc

N = 10000
H = 128
E = 320000

NC = 2
NS = 16
NW = NC * NS
L = 16
C = 128
HL = H // L

NPAD = 10240
ROWS_PT = NPAD // NS
EPAD = 327680
EPT = EPAD // NW
NCHUNK = EPT // C

_mesh = plsc.VectorSubcoreMesh(
    core_axis_name="c", subcore_axis_name="s", num_cores=NC, num_subcores=NS
)


@functools.partial(
    pl.kernel,
    out_type=jax.ShapeDtypeStruct((NC, NPAD, L), jnp.float32),
    mesh=_mesh,
    scratch_types=[
        pltpu.VMEM((C,), jnp.int32),
        pltpu.VMEM((C,), jnp.float32),
        pltpu.VMEM((C, L), jnp.float32),
        pltpu.VMEM_SHARED((NPAD, L), jnp.float32),
    ],
    compiler_params=pltpu.CompilerParams(use_tc_tiling_on_sc=False),
)
def _sc_deg(dst_hbm, w_hbm, out_hbm, dst_v, w_sc, rows_v, acc_sh):
    cid = lax.axis_index("c")
    sid = lax.axis_index("s")
    wid = sid * NC + cid
    lo8 = lax.broadcasted_iota(jnp.int32, (L,), 0) < 8

    def zero_row(i, carry):
        rows_v[i, :] = jnp.zeros((L,), jnp.float32)
        return carry

    lax.fori_loop(0, C, zero_row, 0)
    nbase = sid * ROWS_PT
    for k in range(ROWS_PT // C):
        pltpu.sync_copy(rows_v, acc_sh.at[pl.ds(nbase + k * C, C)])
    plsc.subcore_barrier()

    ebase = wid * EPT

    def chunk(ci, carry):
        off = ebase + ci * C
        pltpu.sync_copy(dst_hbm.at[pl.ds(off, C)], dst_v)
        pltpu.sync_copy(w_hbm.at[pl.ds(off, C)], w_sc)

        @plsc.parallel_loop(0, C // L, unroll=2)
        def _build(g):
            wvec = w_sc[pl.ds(g * L, L)]
            for i in range(L):
                rows_v[g * L + i, :] = jnp.where(lo8, wvec[i],
                                                 jnp.float32(1.0))

        pltpu.sync_copy(rows_v, acc_sh.at[dst_v], add=True)
        return carry

    lax.fori_loop(0, NCHUNK, chunk, 0)
    plsc.subcore_barrier()

    for k in range(ROWS_PT // C):
        pltpu.sync_copy(acc_sh.at[pl.ds(nbase + k * C, C)], rows_v)
        pltpu.sync_copy(rows_v, out_hbm.at[cid, pl.ds(nbase + k * C, C)])


SK = 4
HH = H // 2
HL2 = HH // L
EPT2 = EPAD // NS
NCHUNK2 = EPT2 // C
NSUP = NCHUNK2 // SK
EC = EPAD // C


def _make_agg_body(weighted):
    def body(*args):
        if weighted:
            (yh_hbm, sd_hbm, w_hbm, out_hbm, sd_v, w_v, rows_v, y_sh,
             acc_sh, gsem, ssem) = args
        else:
            (yh_hbm, sd_hbm, out_hbm, sd_v, rows_v, y_sh,
             acc_sh, gsem, ssem) = args
        cid = lax.axis_index("c")
        sid = lax.axis_index("s")

        def zero_row(i, carry):
            for j in range(HL2):
                rows_v[0, i, pl.ds(j * L, L)] = jnp.zeros((L,), jnp.float32)
                rows_v[1, i, pl.ds(j * L, L)] = jnp.zeros((L,), jnp.float32)
            return carry

        lax.fori_loop(0, C, zero_row, 0)
        nbase = sid * ROWS_PT
        for k in range(ROWS_PT // C):
            pltpu.sync_copy(rows_v.at[0], acc_sh.at[pl.ds(nbase + k * C, C)])
            pltpu.sync_copy(yh_hbm.at[cid, pl.ds(nbase + k * C, C)],
                            y_sh.at[pl.ds(nbase + k * C, C)])
        plsc.subcore_barrier()

        rbase = sid * NCHUNK2

        def load_idx(s, b):
            row = rbase + s * SK
            pltpu.sync_copy(sd_hbm.at[pl.ds(row, SK)], sd_v.at[b])
            if weighted:
                pltpu.sync_copy(w_hbm.at[pl.ds(row, SK)], w_v)

        def fire_gather(s, b, j, p):
            pltpu.async_copy(y_sh.at[sd_v.at[b, j, 0]], rows_v.at[p], gsem)

        def wait_gather(b, j, p):
            pltpu.make_async_copy(y_sh.at[sd_v.at[b, j, 0]], rows_v.at[p],
                                  gsem).wait()

        def fire_scatter(b, j, p):
            pltpu.async_copy(rows_v.at[p], acc_sh.at[sd_v.at[b, j, 1]], ssem,
                             add=True)

        def wait_scatter(p):
            pltpu.make_async_copy(rows_v.at[p], acc_sh.at[sd_v.at[0, 0, 1]],
                                  ssem).wait()

        def scale(b, j, p):
            if not weighted:
                return

            @plsc.parallel_loop(0, C // L, unroll=2)
            def _scale(g):
                wvec = w_v[j, pl.ds(g * L, L)]
                for i in range(L):
                    wb = wvec[i]
                    for k in range(HL2):
                        e = g * L + i
                        rows_v[p, e, pl.ds(k * L, L)] = (
                            rows_v[p, e, pl.ds(k * L, L)] * wb)

        load_idx(0, 0)
        fire_gather(0, 0, 0, 0)
        fire_scatter(0, 0, 1)

        def super_chunk(s, carry):
            b = s & 1
            nb = b ^ 1
            last = s == NSUP - 1
            ns = jnp.where(last, s, s + 1)
            for j in range(SK):
                p = j & 1
                wait_gather(b, j, p)
                wait_scatter(p ^ 1)
                if j < SK - 1:
                    fire_gather(s, b, j + 1, p ^ 1)
                    scale(b, j, p)
                    fire_scatter(b, j, p)
                else:
                    scale(b, j, p)
                    fire_scatter(b, j, p)
                    load_idx(ns, nb)
                    fire_gather(ns, nb, 0, p ^ 1)
            return carry

        lax.fori_loop(0, NSUP, super_chunk, 0)
        wait_scatter(0)
        pltpu.make_async_copy(y_sh.at[sd_v.at[0, 0, 0]], rows_v.at[0],
                              gsem).wait()
        plsc.subcore_barrier()

        for k in range(ROWS_PT // C):
            pltpu.sync_copy(acc_sh.at[pl.ds(nbase + k * C, C)], rows_v.at[0])
            pltpu.sync_copy(rows_v.at[0],
                            out_hbm.at[cid, pl.ds(nbase + k * C, C)])

    return body


def _agg_scratch(weighted):
    sc = [
        pltpu.VMEM((2, SK, 2, C), jnp.int32),
        pltpu.VMEM((SK, C), jnp.float32),
        pltpu.VMEM((2, C, HH), jnp.float32),
        pltpu.VMEM_SHARED((NPAD, HH), jnp.float32),
        pltpu.VMEM_SHARED((NPAD, HH), jnp.float32),
        pltpu.SemaphoreType.DMA,
        pltpu.SemaphoreType.DMA,
    ]
    if not weighted:
        del sc[1]
    return sc


_sc_agg_w = pl.kernel(
    _make_agg_body(True),
    out_type=jax.ShapeDtypeStruct((NC, NPAD, HH), jnp.float32),
    mesh=_mesh,
    scratch_types=_agg_scratch(True),
    compiler_params=pltpu.CompilerParams(use_tc_tiling_on_sc=False),
)

_sc_agg_u = pl.kernel(
    _make_agg_body(False),
    out_type=jax.ShapeDtypeStruct((NC, NPAD, HH), jnp.float32),
    mesh=_mesh,
    scratch_types=_agg_scratch(False),
    compiler_params=pltpu.CompilerParams(use_tc_tiling_on_sc=False),
)


def _dis(d_ref):
    return lax.rsqrt(d_ref[0] + d_ref[1] + 1.0)


def _softplus(x):
    return jnp.maximum(x, 0.0) + jnp.log1p(jnp.exp(-jnp.abs(x)))


def _dot(a, b):
    return jnp.dot(a, b, precision=lax.Precision.HIGHEST,
                   preferred_element_type=jnp.float32)


def _halves(r):
    return jnp.concatenate([r[0], r[1]], axis=-1)


def _mm_body(h_ref, w_ref, zh_ref):
    z = _dot(h_ref[...], w_ref[...])
    zh_ref[0] = z[:, :HH]
    zh_ref[1] = z[:, HH:]


def _scale_body(z_ref, d_ref, yh_ref):
    dis = _dis(d_ref)
    yh_ref[0] = dis * z_ref[0]
    yh_ref[1] = dis * z_ref[1]


def _mid_body(p_ref, yh_ref, din_ref, dout_ref, b_ref, w_ref, out_ref):
    x = _dis(din_ref) * (_halves(p_ref) + _halves(yh_ref)) + b_ref[...]
    y = _dis(dout_ref) * _dot(_softplus(x), w_ref[...])
    out_ref[0] = y[:, :HH]
    out_ref[1] = y[:, HH:]


def _post_body(p_ref, yh_ref, d_ref, b_ref, out_ref):
    out_ref[...] = _dis(d_ref) * (_halves(p_ref) + _halves(yh_ref)) + b_ref[...]


_f32 = jnp.float32
_B = 640
_ROW = pl.BlockSpec((_B, H), lambda i: (i, 0))
_HALF = pl.BlockSpec((NC, _B, HH), lambda i: (0, i, 0))
_DEG = pl.BlockSpec((NC, _B, 1), lambda i: (0, i, 0))
_BIAS = pl.BlockSpec((1, H), lambda i: (0, 0))
_WMAT = pl.BlockSpec((H, H), lambda i: (0, 0))
_HSH = jax.ShapeDtypeStruct((NC, NPAD, HH), _f32)

_tc_mm = pl.pallas_call(
    _mm_body, grid=(NPAD // _B,),
    in_specs=[_ROW, _WMAT], out_specs=_HALF, out_shape=_HSH)
_tc_scale = pl.pallas_call(
    _scale_body, grid=(NPAD // _B,),
    in_specs=[_HALF, _DEG], out_specs=_HALF, out_shape=_HSH)
_tc_mid = pl.pallas_call(
    _mid_body, grid=(NPAD // _B,),
    in_specs=[_HALF, _HALF, _DEG, _DEG, _BIAS, _WMAT], out_specs=_HALF,
    out_shape=_HSH)
_tc_post = pl.pallas_call(
    _post_body, grid=(NPAD // _B,),
    in_specs=[_HALF, _HALF, _DEG, _BIAS], out_specs=_ROW,
    out_shape=jax.ShapeDtypeStruct((NPAD, H), _f32))


def kernel(t, x, edge_index, edge_weight, distance_to_root, node_physical_distance,
           kplus, kprimeplus, initial_min, initial_max, W1, b1, W2, b2, W3, b3):
    h0 = jnp.concatenate(
        [x, distance_to_root, node_physical_distance, kplus, kprimeplus,
         initial_min, initial_max], axis=1)
    h0p = jnp.concatenate([h0, jnp.zeros((NPAD - N, H), _f32)])

    pad = EPAD - E
    src_p = jnp.concatenate([edge_index[0], jnp.zeros((pad,), jnp.int32)])
    dst_p = jnp.concatenate([edge_index[1], jnp.full((pad,), N, jnp.int32)])
    w12_1 = jnp.concatenate([edge_weight, jnp.zeros((pad,), _f32)])

    degp = _sc_deg(dst_p, w12_1)
    sd2d = jnp.stack([src_p.reshape(EC, C), dst_p.reshape(EC, C)], axis=1)
    w2d = w12_1.reshape(EC, C)
    d12 = degp[:, :, 0:1]
    d3 = degp[:, :, 8:9]

    b1r, b2r, b3r = (b.reshape(1, H) for b in (b1, b2, b3))

    z1 = _tc_mm(h0p, W1)
    y1 = _tc_scale(z1, d12)
    p1 = _sc_agg_w(y1, sd2d, w2d)
    y2 = _tc_mid(p1, y1, d12, d12, b1r, W2)
    p2 = _sc_agg_w(y2, sd2d, w2d)
    y3 = _tc_mid(p2, y2, d12, d3, b2r, W3)
    p3 = _sc_agg_u(y3, sd2d)
    return _tc_post(p3, y3, d3, b3r)[:N]


# speedup vs baseline: 16.1551x; 1.0084x over previous
import functools

import jax
import jax.numpy as jnp
from jax import lax
from jax.experimental import pallas as pl
from jax.experimental.pallas import tpu as pltpu
from jax.experimental.pallas import tpu_sc as plsc

N = 10000
H = 128
E = 320000

NC = 2
NS = 16
NW = NC * NS
L = 16
C = 128
HL = H // L

NPAD = 10240
ROWS_PT = NPAD // NS
EPAD = 327680
EPT = EPAD // NW
NCHUNK = EPT // C

_mesh = plsc.VectorSubcoreMesh(
    core_axis_name="c", subcore_axis_name="s", num_cores=NC, num_subcores=NS
)


@functools.partial(
    pl.kernel,
    out_type=jax.ShapeDtypeStruct((NC, NPAD, L), jnp.float32),
    mesh=_mesh,
    scratch_types=[
        pltpu.VMEM((2, C), jnp.int32),
        pltpu.VMEM((2, C), jnp.float32),
        pltpu.VMEM((2, C, L), jnp.float32),
        pltpu.VMEM_SHARED((NPAD, L), jnp.float32),
        pltpu.SemaphoreType.DMA,
    ],
    compiler_params=pltpu.CompilerParams(use_tc_tiling_on_sc=False),
)
def _sc_deg(dst_hbm, w_hbm, out_hbm, dst_v, w_sc, rows_v, acc_sh, ssem):
    cid = lax.axis_index("c")
    sid = lax.axis_index("s")
    wid = sid * NC + cid
    lo8 = lax.broadcasted_iota(jnp.int32, (L,), 0) < 8

    def zero_row(i, carry):
        rows_v[0, i, :] = jnp.zeros((L,), jnp.float32)
        rows_v[1, i, :] = jnp.zeros((L,), jnp.float32)
        return carry

    lax.fori_loop(0, C, zero_row, 0)
    for g in range(C // L):
        dst_v[0, pl.ds(g * L, L)] = jnp.zeros((L,), jnp.int32)
        dst_v[1, pl.ds(g * L, L)] = jnp.zeros((L,), jnp.int32)
    nbase = sid * ROWS_PT
    for k in range(ROWS_PT // C):
        pltpu.sync_copy(rows_v.at[0], acc_sh.at[pl.ds(nbase + k * C, C)])
    plsc.subcore_barrier()

    ebase = wid * EPT

    def fire_scatter(b):
        pltpu.async_copy(rows_v.at[b], acc_sh.at[dst_v.at[b]], ssem, add=True)

    def wait_scatter():
        pltpu.make_async_copy(rows_v.at[0], acc_sh.at[dst_v.at[0]],
                              ssem).wait()

    fire_scatter(0)
    fire_scatter(1)

    def chunk(ci, carry):
        b = ci & 1
        off = ebase + ci * C
        wait_scatter()
        pltpu.sync_copy(dst_hbm.at[pl.ds(off, C)], dst_v.at[b])
        pltpu.sync_copy(w_hbm.at[pl.ds(off, C)], w_sc.at[b])

        @plsc.parallel_loop(0, C // L, unroll=2)
        def _build(g):
            wvec = w_sc[b, pl.ds(g * L, L)]
            for i in range(L):
                rows_v[b, g * L + i, :] = jnp.where(lo8, wvec[i],
                                                    jnp.float32(1.0))

        fire_scatter(b)
        return carry

    lax.fori_loop(0, NCHUNK, chunk, 0)
    wait_scatter()
    wait_scatter()
    plsc.subcore_barrier()

    for k in range(ROWS_PT // C):
        pltpu.sync_copy(acc_sh.at[pl.ds(nbase + k * C, C)], rows_v.at[0])
        pltpu.sync_copy(rows_v.at[0], out_hbm.at[cid, pl.ds(nbase + k * C, C)])


SK = 4
HH = H // 2
HL2 = HH // L
EPT2 = EPAD // NS
NCHUNK2 = EPT2 // C
NSUP = NCHUNK2 // SK
EC = EPAD // C


def _make_agg_body(weighted):
    def body(*args):
        if weighted:
            (yh_hbm, sd_hbm, w_hbm, out_hbm, sd_v, w_v, rows_v, y_sh,
             acc_sh, gsem, ssem) = args
        else:
            (yh_hbm, sd_hbm, out_hbm, sd_v, rows_v, y_sh,
             acc_sh, gsem, ssem) = args
        cid = lax.axis_index("c")
        sid = lax.axis_index("s")

        def zero_row(i, carry):
            for j in range(HL2):
                rows_v[0, i, pl.ds(j * L, L)] = jnp.zeros((L,), jnp.float32)
                rows_v[1, i, pl.ds(j * L, L)] = jnp.zeros((L,), jnp.float32)
            return carry

        lax.fori_loop(0, C, zero_row, 0)
        nbase = sid * ROWS_PT
        for k in range(ROWS_PT // C):
            pltpu.sync_copy(rows_v.at[0], acc_sh.at[pl.ds(nbase + k * C, C)])
            pltpu.sync_copy(yh_hbm.at[cid, pl.ds(nbase + k * C, C)],
                            y_sh.at[pl.ds(nbase + k * C, C)])
        plsc.subcore_barrier()

        rbase = sid * NCHUNK2

        def load_idx(s, b):
            row = rbase + s * SK
            pltpu.sync_copy(sd_hbm.at[pl.ds(row, SK)], sd_v.at[b])
            if weighted:
                pltpu.sync_copy(w_hbm.at[pl.ds(row, SK)], w_v)

        def fire_gather(s, b, j, p):
            pltpu.async_copy(y_sh.at[sd_v.at[b, j, 0]], rows_v.at[p], gsem)

        def wait_gather(b, j, p):
            pltpu.make_async_copy(y_sh.at[sd_v.at[b, j, 0]], rows_v.at[p],
                                  gsem).wait()

        def fire_scatter(b, j, p):
            pltpu.async_copy(rows_v.at[p], acc_sh.at[sd_v.at[b, j, 1]], ssem,
                             add=True)

        def wait_scatter(p):
            pltpu.make_async_copy(rows_v.at[p], acc_sh.at[sd_v.at[0, 0, 1]],
                                  ssem).wait()

        def scale(b, j, p):
            if not weighted:
                return

            @plsc.parallel_loop(0, C // L, unroll=2)
            def _scale(g):
                wvec = w_v[j, pl.ds(g * L, L)]
                for i in range(L):
                    wb = wvec[i]
                    for k in range(HL2):
                        e = g * L + i
                        rows_v[p, e, pl.ds(k * L, L)] = (
                            rows_v[p, e, pl.ds(k * L, L)] * wb)

        load_idx(0, 0)
        fire_gather(0, 0, 0, 0)
        fire_scatter(0, 0, 1)

        def super_chunk(s, carry):
            b = s & 1
            nb = b ^ 1
            last = s == NSUP - 1
            ns = jnp.where(last, s, s + 1)
            for j in range(SK):
                p = j & 1
                wait_gather(b, j, p)
                wait_scatter(p ^ 1)
                if j < SK - 1:
                    fire_gather(s, b, j + 1, p ^ 1)
                    scale(b, j, p)
                    fire_scatter(b, j, p)
                else:
                    scale(b, j, p)
                    fire_scatter(b, j, p)
                    load_idx(ns, nb)
                    fire_gather(ns, nb, 0, p ^ 1)
            return carry

        lax.fori_loop(0, NSUP, super_chunk, 0)
        wait_scatter(0)
        pltpu.make_async_copy(y_sh.at[sd_v.at[0, 0, 0]], rows_v.at[0],
                              gsem).wait()
        plsc.subcore_barrier()

        for k in range(ROWS_PT // C):
            pltpu.sync_copy(acc_sh.at[pl.ds(nbase + k * C, C)], rows_v.at[0])
            pltpu.sync_copy(rows_v.at[0],
                            out_hbm.at[cid, pl.ds(nbase + k * C, C)])

    return body


def _agg_scratch(weighted):
    sc = [
        pltpu.VMEM((2, SK, 2, C), jnp.int32),
        pltpu.VMEM((SK, C), jnp.float32),
        pltpu.VMEM((2, C, HH), jnp.float32),
        pltpu.VMEM_SHARED((NPAD, HH), jnp.float32),
        pltpu.VMEM_SHARED((NPAD, HH), jnp.float32),
        pltpu.SemaphoreType.DMA,
        pltpu.SemaphoreType.DMA,
    ]
    if not weighted:
        del sc[1]
    return sc


_sc_agg_w = pl.kernel(
    _make_agg_body(True),
    out_type=jax.ShapeDtypeStruct((NC, NPAD, HH), jnp.float32),
    mesh=_mesh,
    scratch_types=_agg_scratch(True),
    compiler_params=pltpu.CompilerParams(use_tc_tiling_on_sc=False),
)

_sc_agg_u = pl.kernel(
    _make_agg_body(False),
    out_type=jax.ShapeDtypeStruct((NC, NPAD, HH), jnp.float32),
    mesh=_mesh,
    scratch_types=_agg_scratch(False),
    compiler_params=pltpu.CompilerParams(use_tc_tiling_on_sc=False),
)


def _dis(d_ref):
    return lax.rsqrt(d_ref[0] + d_ref[1] + 1.0)


def _softplus(x):
    return jnp.maximum(x, 0.0) + jnp.log1p(jnp.exp(-jnp.abs(x)))


def _dot(a, b):
    return jnp.dot(a, b, precision=lax.Precision.HIGHEST,
                   preferred_element_type=jnp.float32)


def _halves(r):
    return jnp.concatenate([r[0], r[1]], axis=-1)


def _mm_body(h_ref, w_ref, zh_ref):
    z = _dot(h_ref[...], w_ref[...])
    zh_ref[0] = z[:, :HH]
    zh_ref[1] = z[:, HH:]


def _scale_body(z_ref, d_ref, yh_ref):
    dis = _dis(d_ref)
    yh_ref[0] = dis * z_ref[0]
    yh_ref[1] = dis * z_ref[1]


def _mid_body(p_ref, yh_ref, din_ref, dout_ref, b_ref, w_ref, out_ref):
    x = _dis(din_ref) * (_halves(p_ref) + _halves(yh_ref)) + b_ref[...]
    y = _dis(dout_ref) * _dot(_softplus(x), w_ref[...])
    out_ref[0] = y[:, :HH]
    out_ref[1] = y[:, HH:]


def _post_body(p_ref, yh_ref, d_ref, b_ref, out_ref):
    out_ref[...] = _dis(d_ref) * (_halves(p_ref) + _halves(yh_ref)) + b_ref[...]


_f32 = jnp.float32
_B = 640
_ROW = pl.BlockSpec((_B, H), lambda i: (i, 0))
_HALF = pl.BlockSpec((NC, _B, HH), lambda i: (0, i, 0))
_DEG = pl.BlockSpec((NC, _B, 1), lambda i: (0, i, 0))
_BIAS = pl.BlockSpec((1, H), lambda i: (0, 0))
_WMAT = pl.BlockSpec((H, H), lambda i: (0, 0))
_HSH = jax.ShapeDtypeStruct((NC, NPAD, HH), _f32)

_tc_mm = pl.pallas_call(
    _mm_body, grid=(NPAD // _B,),
    in_specs=[_ROW, _WMAT], out_specs=_HALF, out_shape=_HSH)
_tc_scale = pl.pallas_call(
    _scale_body, grid=(NPAD // _B,),
    in_specs=[_HALF, _DEG], out_specs=_HALF, out_shape=_HSH)
_tc_mid = pl.pallas_call(
    _mid_body, grid=(NPAD // _B,),
    in_specs=[_HALF, _HALF, _DEG, _DEG, _BIAS, _WMAT], out_specs=_HALF,
    out_shape=_HSH)
_tc_post = pl.pallas_call(
    _post_body, grid=(NPAD // _B,),
    in_specs=[_HALF, _HALF, _DEG, _BIAS], out_specs=_ROW,
    out_shape=jax.ShapeDtypeStruct((NPAD, H), _f32))


def kernel(t, x, edge_index, edge_weight, distance_to_root, node_physical_distance,
           kplus, kprimeplus, initial_min, initial_max, W1, b1, W2, b2, W3, b3):
    h0 = jnp.concatenate(
        [x, distance_to_root, node_physical_distance, kplus, kprimeplus,
         initial_min, initial_max], axis=1)
    h0p = jnp.concatenate([h0, jnp.zeros((NPAD - N, H), _f32)])

    pad = EPAD - E
    src_p = jnp.concatenate([edge_index[0], jnp.zeros((pad,), jnp.int32)])
    dst_p = jnp.concatenate([edge_index[1], jnp.full((pad,), N, jnp.int32)])
    w12_1 = jnp.concatenate([edge_weight, jnp.zeros((pad,), _f32)])

    degp = _sc_deg(dst_p, w12_1)
    sd2d = jnp.stack([src_p.reshape(EC, C), dst_p.reshape(EC, C)], axis=1)
    w2d = w12_1.reshape(EC, C)
    d12 = degp[:, :, 0:1]
    d3 = degp[:, :, 8:9]

    b1r, b2r, b3r = (b.reshape(1, H) for b in (b1, b2, b3))

    z1 = _tc_mm(h0p, W1)
    y1 = _tc_scale(z1, d12)
    p1 = _sc_agg_w(y1, sd2d, w2d)
    y2 = _tc_mid(p1, y1, d12, d12, b1r, W2)
    p2 = _sc_agg_w(y2, sd2d, w2d)
    y3 = _tc_mid(p2, y2, d12, d3, b2r, W3)
    p3 = _sc_agg_u(y3, sd2d)
    return _tc_post(p3, y3, d3, b3r)[:N]


# speedup vs baseline: 19.4868x; 1.2062x over previous
import functools

import jax
import jax.numpy as jnp
from jax import lax
from jax.experimental import pallas as pl
from jax.experimental.pallas import tpu as pltpu
from jax.experimental.pallas import tpu_sc as plsc

N = 10000
H = 128
E = 320000

NC = 2
NS = 16
NW = NC * NS
L = 16
C = 128
HL = H // L

NPAD = 10240
ROWS_PT = NPAD // NS
EPAD = 327680
EPT = EPAD // NW
NCHUNK = EPT // C

_mesh = plsc.VectorSubcoreMesh(
    core_axis_name="c", subcore_axis_name="s", num_cores=NC, num_subcores=NS
)


@functools.partial(
    pl.kernel,
    out_type=jax.ShapeDtypeStruct((NC, NPAD, L), jnp.float32),
    mesh=_mesh,
    scratch_types=[
        pltpu.VMEM((2, C), jnp.int32),
        pltpu.VMEM((2, C), jnp.float32),
        pltpu.VMEM((2, C, L), jnp.float32),
        pltpu.VMEM_SHARED((NPAD, L), jnp.float32),
        pltpu.SemaphoreType.DMA,
    ],
    compiler_params=pltpu.CompilerParams(use_tc_tiling_on_sc=False),
)
def _sc_deg(dst_hbm, w_hbm, out_hbm, dst_v, w_sc, rows_v, acc_sh, ssem):
    cid = lax.axis_index("c")
    sid = lax.axis_index("s")
    wid = sid * NC + cid
    lo8 = lax.broadcasted_iota(jnp.int32, (L,), 0) < 8

    def zero_row(i, carry):
        rows_v[0, i, :] = jnp.zeros((L,), jnp.float32)
        rows_v[1, i, :] = jnp.zeros((L,), jnp.float32)
        return carry

    lax.fori_loop(0, C, zero_row, 0)
    for g in range(C // L):
        dst_v[0, pl.ds(g * L, L)] = jnp.zeros((L,), jnp.int32)
        dst_v[1, pl.ds(g * L, L)] = jnp.zeros((L,), jnp.int32)
    nbase = sid * ROWS_PT
    for k in range(ROWS_PT // C):
        pltpu.sync_copy(rows_v.at[0], acc_sh.at[pl.ds(nbase + k * C, C)])
    plsc.subcore_barrier()

    ebase = wid * EPT

    def fire_scatter(b):
        pltpu.async_copy(rows_v.at[b], acc_sh.at[dst_v.at[b]], ssem, add=True)

    def wait_scatter():
        pltpu.make_async_copy(rows_v.at[0], acc_sh.at[dst_v.at[0]],
                              ssem).wait()

    fire_scatter(0)
    fire_scatter(1)

    def chunk(ci, carry):
        b = ci & 1
        off = ebase + ci * C
        wait_scatter()
        pltpu.sync_copy(dst_hbm.at[pl.ds(off, C)], dst_v.at[b])
        pltpu.sync_copy(w_hbm.at[pl.ds(off, C)], w_sc.at[b])

        @plsc.parallel_loop(0, C // L, unroll=2)
        def _build(g):
            wvec = w_sc[b, pl.ds(g * L, L)]
            for i in range(L):
                rows_v[b, g * L + i, :] = jnp.where(lo8, wvec[i],
                                                    jnp.float32(1.0))

        fire_scatter(b)
        return carry

    lax.fori_loop(0, NCHUNK, chunk, 0)
    wait_scatter()
    wait_scatter()
    plsc.subcore_barrier()

    for k in range(ROWS_PT // C):
        pltpu.sync_copy(acc_sh.at[pl.ds(nbase + k * C, C)], rows_v.at[0])
        pltpu.sync_copy(rows_v.at[0], out_hbm.at[cid, pl.ds(nbase + k * C, C)])


SK = 4
HH = H // 2
HL2 = HH // L
EPT2 = EPAD // NS
NCHUNK2 = EPT2 // C
NSUP = NCHUNK2 // SK
EC = EPAD // C


def _make_agg_body(weighted):
    def body(*args):
        if weighted:
            (yh_hbm, sd_hbm, w_hbm, out_hbm, sd_v, w_v, rows_v, y_sh,
             acc_sh, gsem, ssem) = args
        else:
            (yh_hbm, sd_hbm, out_hbm, sd_v, rows_v, y_sh,
             acc_sh, gsem, ssem) = args
        cid = lax.axis_index("c")
        sid = lax.axis_index("s")

        def zero_row(i, carry):
            for j in range(HL2):
                for q in range(4):
                    rows_v[q, i, pl.ds(j * L, L)] = jnp.zeros((L,),
                                                              jnp.float32)
            return carry

        lax.fori_loop(0, C, zero_row, 0)
        nbase = sid * ROWS_PT
        for k in range(ROWS_PT // C):
            pltpu.sync_copy(rows_v.at[0], acc_sh.at[pl.ds(nbase + k * C, C)])
            pltpu.sync_copy(yh_hbm.at[cid, pl.ds(nbase + k * C, C)],
                            y_sh.at[pl.ds(nbase + k * C, C)])
        plsc.subcore_barrier()

        rbase = sid * NCHUNK2

        def load_sd(s, b):
            row = rbase + s * SK
            pltpu.sync_copy(sd_hbm.at[pl.ds(row, SK)], sd_v.at[b])

        def load_w(s):
            if weighted:
                row = rbase + s * SK
                pltpu.sync_copy(w_hbm.at[pl.ds(row, SK)], w_v)

        def fire_gather(s, b, j, p):
            pltpu.async_copy(y_sh.at[sd_v.at[b, j, 0]], rows_v.at[p], gsem)

        def wait_gather(b, j, p):
            pltpu.make_async_copy(y_sh.at[sd_v.at[b, j, 0]], rows_v.at[p],
                                  gsem).wait()

        def fire_scatter(b, j, p):
            pltpu.async_copy(rows_v.at[p], acc_sh.at[sd_v.at[b, j, 1]], ssem,
                             add=True)

        def wait_scatter(p):
            pltpu.make_async_copy(rows_v.at[p], acc_sh.at[sd_v.at[0, 0, 1]],
                                  ssem).wait()

        def scale(b, j, p):
            if not weighted:
                return

            @plsc.parallel_loop(0, C // L, unroll=2)
            def _scale(g):
                wvec = w_v[j, pl.ds(g * L, L)]
                for i in range(L):
                    wb = wvec[i]
                    for k in range(HL2):
                        e = g * L + i
                        rows_v[p, e, pl.ds(k * L, L)] = (
                            rows_v[p, e, pl.ds(k * L, L)] * wb)

        load_sd(0, 0)
        load_w(0)
        fire_gather(0, 0, 0, 0)
        fire_gather(0, 0, 1, 1)
        fire_scatter(0, 0, 2)
        fire_scatter(0, 0, 3)

        def super_chunk(s, carry):
            b = s & 1
            nb = b ^ 1
            last = s == NSUP - 1
            ns = jnp.where(last, s, s + 1)
            for j in range(SK):
                wait_gather(b, j, j)
                wait_scatter(j)
                if j < SK - 2:
                    fire_gather(s, b, j + 2, j + 2)
                else:
                    fire_gather(ns, nb, j - 2, j - 2)
                scale(b, j, j)
                fire_scatter(b, j, j)
                if j == 1:
                    load_sd(ns, nb)
                if j == SK - 1:
                    load_w(ns)
            return carry

        lax.fori_loop(0, NSUP, super_chunk, 0)
        wait_scatter(0)
        wait_scatter(0)
        pltpu.make_async_copy(y_sh.at[sd_v.at[0, 0, 0]], rows_v.at[0],
                              gsem).wait()
        pltpu.make_async_copy(y_sh.at[sd_v.at[0, 0, 0]], rows_v.at[1],
                              gsem).wait()
        plsc.subcore_barrier()

        for k in range(ROWS_PT // C):
            pltpu.sync_copy(acc_sh.at[pl.ds(nbase + k * C, C)], rows_v.at[0])
            pltpu.sync_copy(rows_v.at[0],
                            out_hbm.at[cid, pl.ds(nbase + k * C, C)])

    return body


def _agg_scratch(weighted):
    sc = [
        pltpu.VMEM((2, SK, 2, C), jnp.int32),
        pltpu.VMEM((SK, C), jnp.float32),
        pltpu.VMEM((4, C, HH), jnp.float32),
        pltpu.VMEM_SHARED((NPAD, HH), jnp.float32),
        pltpu.VMEM_SHARED((NPAD, HH), jnp.float32),
        pltpu.SemaphoreType.DMA,
        pltpu.SemaphoreType.DMA,
    ]
    if not weighted:
        del sc[1]
    return sc


_sc_agg_w = pl.kernel(
    _make_agg_body(True),
    out_type=jax.ShapeDtypeStruct((NC, NPAD, HH), jnp.float32),
    mesh=_mesh,
    scratch_types=_agg_scratch(True),
    compiler_params=pltpu.CompilerParams(use_tc_tiling_on_sc=False),
)

_sc_agg_u = pl.kernel(
    _make_agg_body(False),
    out_type=jax.ShapeDtypeStruct((NC, NPAD, HH), jnp.float32),
    mesh=_mesh,
    scratch_types=_agg_scratch(False),
    compiler_params=pltpu.CompilerParams(use_tc_tiling_on_sc=False),
)


def _dis(d_ref):
    return lax.rsqrt(d_ref[0] + d_ref[1] + 1.0)


def _softplus(x):
    return jnp.maximum(x, 0.0) + jnp.log1p(jnp.exp(-jnp.abs(x)))


def _dot(a, b):
    return jnp.dot(a, b, precision=lax.Precision.HIGHEST,
                   preferred_element_type=jnp.float32)


def _halves(r):
    return jnp.concatenate([r[0], r[1]], axis=-1)


def _mm_body(h_ref, w_ref, zh_ref):
    z = _dot(h_ref[...], w_ref[...])
    zh_ref[0] = z[:, :HH]
    zh_ref[1] = z[:, HH:]


def _scale_body(z_ref, d_ref, yh_ref):
    dis = _dis(d_ref)
    yh_ref[0] = dis * z_ref[0]
    yh_ref[1] = dis * z_ref[1]


def _mid_body(p_ref, yh_ref, din_ref, dout_ref, b_ref, w_ref, out_ref):
    x = _dis(din_ref) * (_halves(p_ref) + _halves(yh_ref)) + b_ref[...]
    y = _dis(dout_ref) * _dot(_softplus(x), w_ref[...])
    out_ref[0] = y[:, :HH]
    out_ref[1] = y[:, HH:]


def _post_body(p_ref, yh_ref, d_ref, b_ref, out_ref):
    out_ref[...] = _dis(d_ref) * (_halves(p_ref) + _halves(yh_ref)) + b_ref[...]


_f32 = jnp.float32
_B = 640
_ROW = pl.BlockSpec((_B, H), lambda i: (i, 0))
_HALF = pl.BlockSpec((NC, _B, HH), lambda i: (0, i, 0))
_DEG = pl.BlockSpec((NC, _B, 1), lambda i: (0, i, 0))
_BIAS = pl.BlockSpec((1, H), lambda i: (0, 0))
_WMAT = pl.BlockSpec((H, H), lambda i: (0, 0))
_HSH = jax.ShapeDtypeStruct((NC, NPAD, HH), _f32)

_tc_mm = pl.pallas_call(
    _mm_body, grid=(NPAD // _B,),
    in_specs=[_ROW, _WMAT], out_specs=_HALF, out_shape=_HSH)
_tc_scale = pl.pallas_call(
    _scale_body, grid=(NPAD // _B,),
    in_specs=[_HALF, _DEG], out_specs=_HALF, out_shape=_HSH)
_tc_mid = pl.pallas_call(
    _mid_body, grid=(NPAD // _B,),
    in_specs=[_HALF, _HALF, _DEG, _DEG, _BIAS, _WMAT], out_specs=_HALF,
    out_shape=_HSH)
_tc_post = pl.pallas_call(
    _post_body, grid=(NPAD // _B,),
    in_specs=[_HALF, _HALF, _DEG, _BIAS], out_specs=_ROW,
    out_shape=jax.ShapeDtypeStruct((NPAD, H), _f32))


def kernel(t, x, edge_index, edge_weight, distance_to_root, node_physical_distance,
           kplus, kprimeplus, initial_min, initial_max, W1, b1, W2, b2, W3, b3):
    h0 = jnp.concatenate(
        [x, distance_to_root, node_physical_distance, kplus, kprimeplus,
         initial_min, initial_max], axis=1)
    h0p = jnp.concatenate([h0, jnp.zeros((NPAD - N, H), _f32)])

    pad = EPAD - E
    src_p = jnp.concatenate([edge_index[0], jnp.zeros((pad,), jnp.int32)])
    dst_p = jnp.concatenate([edge_index[1], jnp.full((pad,), N, jnp.int32)])
    w12_1 = jnp.concatenate([edge_weight, jnp.zeros((pad,), _f32)])

    degp = _sc_deg(dst_p, w12_1)
    sd2d = jnp.stack([src_p.reshape(EC, C), dst_p.reshape(EC, C)], axis=1)
    w2d = w12_1.reshape(EC, C)
    d12 = degp[:, :, 0:1]
    d3 = degp[:, :, 8:9]

    b1r, b2r, b3r = (b.reshape(1, H) for b in (b1, b2, b3))

    z1 = _tc_mm(h0p, W1)
    y1 = _tc_scale(z1, d12)
    p1 = _sc_agg_w(y1, sd2d, w2d)
    y2 = _tc_mid(p1, y1, d12, d12, b1r, W2)
    p2 = _sc_agg_w(y2, sd2d, w2d)
    y3 = _tc_mid(p2, y2, d12, d3, b2r, W3)
    p3 = _sc_agg_u(y3, sd2d)
    return _tc_post(p3, y3, d3, b3r)[:N]
